# baseline (speedup 1.0000x reference)
import functools

import jax
import jax.numpy as jnp
from jax import lax
from jax.experimental import pallas as pl
from jax.experimental.pallas import tpu as pltpu
from jax.experimental.pallas import tpu_sc as plsc

_B, _S, _F = 1024, 64, 19
_DM = 1152
_D9 = _DM // 9
_D6 = _D9 * 6
_ROWS = _B * _S
_BLOCK_ROWS = 2048

_NC, _NS = 2, 16
_NW = _NC * _NS
_BPW = _ROWS // _NW
_CH = 128
_NCH = _BPW // _CH
_L = 16


def _sc_gather_body(dd_hbm, pl_hbm, mg_hbm, tdd_hbm, tpl_hbm, tmg_hbm,
                    g_hbm, dd_v, pl_v, mg_v, tdd_v, tpl_v, tmg_v,
                    out_a, out_b, sem_a, sem_b):
    wid = lax.axis_index("s") * _NC + lax.axis_index("c")
    base = wid * _BPW
    pltpu.sync_copy(dd_hbm.at[pl.ds(base, _BPW)], dd_v)
    pltpu.sync_copy(pl_hbm.at[pl.ds(base, _BPW)], pl_v)
    pltpu.sync_copy(mg_hbm.at[pl.ds(base, _BPW)], mg_v)
    pltpu.sync_copy(tdd_hbm, tdd_v)
    pltpu.sync_copy(tpl_hbm, tpl_v)
    pltpu.sync_copy(tmg_hbm, tmg_v)

    bufs = (out_a, out_b)
    sems = (sem_a, sem_b)
    lanes = lax.broadcasted_iota(jnp.int32, (_L,), 0)
    handles = [None, None]
    for c in range(_NCH):
        buf = bufs[c % 2]
        if handles[c % 2] is not None:
            handles[c % 2].wait()

        groups = []
        for grp in range(_CH // _L):
            row0 = c * _CH + grp * _L
            dd16 = dd_v[pl.ds(row0, _L)]
            pl16 = pl_v[pl.ds(row0, _L)]
            mg16 = mg_v[pl.ds(row0, _L)]
            rows = lanes + grp * _L
            groups.append((dd16, pl16, mg16, rows))

        def col_body(j, jv, groups=groups, buf=buf):
            for dd16, pl16, mg16, rows in groups:
                v1 = plsc.load_gather(tdd_v, [dd16, jv])
                v2 = plsc.load_gather(tpl_v, [pl16, jv])
                v3 = plsc.load_gather(tmg_v, [mg16, jv])
                plsc.store_scatter(buf, [rows, jv], v1)
                plsc.store_scatter(buf, [rows, jv + _D9], v2)
                plsc.store_scatter(buf, [rows, jv + 2 * _D9], v3)
            return jv + 1

        lax.fori_loop(0, _D9, col_body, jnp.zeros((_L,), jnp.int32))
        handles[c % 2] = pltpu.async_copy(
            buf, g_hbm.at[pl.ds(base + c * _CH, _CH)], sems[c % 2])
    for h in handles:
        if h is not None:
            h.wait()


@functools.partial(
    pl.kernel,
    out_type=jax.ShapeDtypeStruct((_ROWS, 3 * _D9), jnp.float32),
    mesh=plsc.VectorSubcoreMesh(core_axis_name="c", subcore_axis_name="s",
                                num_cores=_NC, num_subcores=_NS),
    compiler_params=pltpu.CompilerParams(needs_layout_passes=False),
    scratch_types=[
        pltpu.VMEM((_BPW,), jnp.int32),
        pltpu.VMEM((_BPW,), jnp.int32),
        pltpu.VMEM((_BPW,), jnp.int32),
        pltpu.VMEM((20, _D9), jnp.float32),
        pltpu.VMEM((64, _D9), jnp.float32),
        pltpu.VMEM((20, _D9), jnp.float32),
        pltpu.VMEM((_CH, 3 * _D9), jnp.float32),
        pltpu.VMEM((_CH, 3 * _D9), jnp.float32),
        pltpu.SemaphoreType.DMA,
        pltpu.SemaphoreType.DMA,
    ],
)
def _sc_gather(*args):
    _sc_gather_body(*args)


def _asm_kernel(x_ref, w_ref, b_ref, pos_ref, g_ref, o_ref):
    xb = x_ref[...]
    cont = xb[:, 0:_F - 3]
    x1 = jax.lax.dot_general(
        cont, w_ref[...], (((1,), (0,)), ((), ())),
        preferred_element_type=jnp.float32) + b_ref[...]
    y = jnp.concatenate([x1, g_ref[...].reshape(_BLOCK_ROWS, 3 * _D9)],
                        axis=1)
    y = y.reshape(_BLOCK_ROWS // _S, _S, _DM) + pos_ref[...][None]
    o_ref[...] = y.reshape(_BLOCK_ROWS, _DM)


def kernel(x, W, b, tab_dd, tab_plate, tab_magtype, tab_pos):
    x2d = x.reshape(_ROWS, _F)
    pl_i = jnp.clip(x2d[:, _F - 3].astype(jnp.int32), 0, 63)
    dd_i = jnp.clip(x2d[:, _F - 2].astype(jnp.int32), 0, 19)
    mg_i = jnp.clip(x2d[:, _F - 1].astype(jnp.int32), 0, 19)

    g = _sc_gather(dd_i, pl_i, mg_i, tab_dd, tab_plate, tab_magtype)

    b2d = b.reshape(1, _D6)
    grid = (_ROWS // _BLOCK_ROWS,)
    out = pl.pallas_call(
        _asm_kernel,
        grid=grid,
        in_specs=[
            pl.BlockSpec((_BLOCK_ROWS, _F), lambda i: (i, 0)),
            pl.BlockSpec((_F - 3, _D6), lambda i: (0, 0)),
            pl.BlockSpec((1, _D6), lambda i: (0, 0)),
            pl.BlockSpec((_S, _DM), lambda i: (0, 0)),
            pl.BlockSpec((_BLOCK_ROWS, 3 * _D9), lambda i: (i, 0)),
        ],
        out_specs=pl.BlockSpec((_BLOCK_ROWS, _DM), lambda i: (i, 0)),
        out_shape=jax.ShapeDtypeStruct((_ROWS, _DM), jnp.float32),
        compiler_params=pltpu.CompilerParams(
            dimension_semantics=("arbitrary",)),
    )(x2d, W, b2d, tab_pos, g)
    return out.reshape(_B, _S, _DM)


# speedup vs baseline: 1.3688x; 1.3688x over previous
import functools

import jax
import jax.numpy as jnp
from jax import lax
from jax.experimental import pallas as pl
from jax.experimental.pallas import tpu as pltpu
from jax.experimental.pallas import tpu_sc as plsc

_B, _S, _F = 1024, 64, 19
_DM = 1152
_D9 = _DM // 9
_D6 = _D9 * 6
_ROWS = _B * _S
_BLOCK_ROWS = 2048

_NC, _NS = 2, 16
_NW = _NC * _NS
_BPW = _ROWS // _NW
_CH = 128
_NCH = _BPW // _CH
_L = 16


def _sc_gather_body(dd_hbm, pl_hbm, mg_hbm, tdd_hbm, tpl_hbm, tmg_hbm,
                    g_hbm, dd_v, pl_v, mg_v, tdd_v, tpl_v, tmg_v,
                    out_a, out_b, sem_a, sem_b):
    wid = lax.axis_index("s") * _NC + lax.axis_index("c")
    base = wid * _BPW
    pltpu.sync_copy(dd_hbm.at[pl.ds(base, _BPW)], dd_v)
    pltpu.sync_copy(pl_hbm.at[pl.ds(base, _BPW)], pl_v)
    pltpu.sync_copy(mg_hbm.at[pl.ds(base, _BPW)], mg_v)
    pltpu.sync_copy(tdd_hbm, tdd_v)
    pltpu.sync_copy(tpl_hbm, tpl_v)
    pltpu.sync_copy(tmg_hbm, tmg_v)

    bufs = (out_a, out_b)
    sems = (sem_a, sem_b)
    lanes = lax.broadcasted_iota(jnp.int32, (_L,), 0)
    handles = [None, None]
    for c in range(_NCH):
        buf = bufs[c % 2]
        if handles[c % 2] is not None:
            handles[c % 2].wait()

        groups = []
        for grp in range(_CH // _L):
            row0 = c * _CH + grp * _L
            dd16 = dd_v[pl.ds(row0, _L)]
            pl16 = pl_v[pl.ds(row0, _L)]
            mg16 = mg_v[pl.ds(row0, _L)]
            rows = lanes + grp * _L
            groups.append((dd16, pl16, mg16, rows))

        @plsc.parallel_loop(0, _D9, unroll=4,
                            carry=jnp.zeros((_L,), jnp.int32))
        def _col_loop(j, jv, groups=groups, buf=buf):
            for dd16, pl16, mg16, rows in groups:
                v1 = plsc.load_gather(tdd_v, [dd16, jv])
                v2 = plsc.load_gather(tpl_v, [pl16, jv])
                v3 = plsc.load_gather(tmg_v, [mg16, jv])
                plsc.store_scatter(buf, [rows, jv], v1)
                plsc.store_scatter(buf, [rows, jv + _D9], v2)
                plsc.store_scatter(buf, [rows, jv + 2 * _D9], v3)
            return jv + 1
        handles[c % 2] = pltpu.async_copy(
            buf, g_hbm.at[pl.ds(base + c * _CH, _CH)], sems[c % 2])
    for h in handles:
        if h is not None:
            h.wait()


@functools.partial(
    pl.kernel,
    out_type=jax.ShapeDtypeStruct((_ROWS, 3 * _D9), jnp.float32),
    mesh=plsc.VectorSubcoreMesh(core_axis_name="c", subcore_axis_name="s",
                                num_cores=_NC, num_subcores=_NS),
    compiler_params=pltpu.CompilerParams(needs_layout_passes=False),
    scratch_types=[
        pltpu.VMEM((_BPW,), jnp.int32),
        pltpu.VMEM((_BPW,), jnp.int32),
        pltpu.VMEM((_BPW,), jnp.int32),
        pltpu.VMEM((20, _D9), jnp.float32),
        pltpu.VMEM((64, _D9), jnp.float32),
        pltpu.VMEM((20, _D9), jnp.float32),
        pltpu.VMEM((_CH, 3 * _D9), jnp.float32),
        pltpu.VMEM((_CH, 3 * _D9), jnp.float32),
        pltpu.SemaphoreType.DMA,
        pltpu.SemaphoreType.DMA,
    ],
)
def _sc_gather(*args):
    _sc_gather_body(*args)


def _asm_kernel(x_ref, w_ref, b_ref, pos_ref, g_ref, o_ref):
    xb = x_ref[...]
    cont = xb[:, 0:_F - 3]
    x1 = jax.lax.dot_general(
        cont, w_ref[...], (((1,), (0,)), ((), ())),
        preferred_element_type=jnp.float32) + b_ref[...]
    y = jnp.concatenate([x1, g_ref[...].reshape(_BLOCK_ROWS, 3 * _D9)],
                        axis=1)
    y = y.reshape(_BLOCK_ROWS // _S, _S, _DM) + pos_ref[...][None]
    o_ref[...] = y.reshape(_BLOCK_ROWS, _DM)


def kernel(x, W, b, tab_dd, tab_plate, tab_magtype, tab_pos):
    x2d = x.reshape(_ROWS, _F)
    pl_i = jnp.clip(x2d[:, _F - 3].astype(jnp.int32), 0, 63)
    dd_i = jnp.clip(x2d[:, _F - 2].astype(jnp.int32), 0, 19)
    mg_i = jnp.clip(x2d[:, _F - 1].astype(jnp.int32), 0, 19)

    g = _sc_gather(dd_i, pl_i, mg_i, tab_dd, tab_plate, tab_magtype)

    b2d = b.reshape(1, _D6)
    grid = (_ROWS // _BLOCK_ROWS,)
    out = pl.pallas_call(
        _asm_kernel,
        grid=grid,
        in_specs=[
            pl.BlockSpec((_BLOCK_ROWS, _F), lambda i: (i, 0)),
            pl.BlockSpec((_F - 3, _D6), lambda i: (0, 0)),
            pl.BlockSpec((1, _D6), lambda i: (0, 0)),
            pl.BlockSpec((_S, _DM), lambda i: (0, 0)),
            pl.BlockSpec((_BLOCK_ROWS, 3 * _D9), lambda i: (i, 0)),
        ],
        out_specs=pl.BlockSpec((_BLOCK_ROWS, _DM), lambda i: (i, 0)),
        out_shape=jax.ShapeDtypeStruct((_ROWS, _DM), jnp.float32),
        compiler_params=pltpu.CompilerParams(
            dimension_semantics=("arbitrary",)),
    )(x2d, W, b2d, tab_pos, g)
    return out.reshape(_B, _S, _DM)


# speedup vs baseline: 8.0664x; 5.8929x over previous
import jax
import jax.numpy as jnp
from jax.experimental import pallas as pl
from jax.experimental.pallas import tpu as pltpu

_B, _S, _F = 1024, 64, 19
_DM = 1152
_D9 = _DM // 9
_D6 = _D9 * 6
_ROWS = _B * _S
_BLOCK_ROWS = 4096


def _pe_kernel(x_ref, w_ref, b_ref, tcat_ref, pos_ref, o_ref):
    xb = x_ref[...]
    cont = xb[:, 0:_F - 3]

    plate = xb[:, _F - 3:_F - 2].astype(jnp.int32)
    dd = xb[:, _F - 2:_F - 1].astype(jnp.int32)
    mag = xb[:, _F - 1:_F].astype(jnp.int32)
    plate = jnp.clip(plate, 0, 63)
    dd = jnp.clip(dd, 0, 19)
    mag = jnp.clip(mag, 0, 19)

    j = jax.lax.broadcasted_iota(jnp.int32, (1, _D9), 1)
    oh = ((dd == j).astype(jnp.float32)
          + (plate == j - 20).astype(jnp.float32)
          + (mag == j - 84).astype(jnp.float32))

    x1 = jax.lax.dot_general(
        cont, w_ref[...], (((1,), (0,)), ((), ())),
        preferred_element_type=jnp.float32) + b_ref[...]
    x234 = jax.lax.dot_general(
        oh, tcat_ref[...], (((1,), (0,)), ((), ())),
        preferred_element_type=jnp.float32)

    y = jnp.concatenate([x1, x234], axis=1)
    y = y.reshape(_BLOCK_ROWS // _S, _S, _DM) + pos_ref[...][None]
    o_ref[...] = y.reshape(_BLOCK_ROWS, _DM)


def kernel(x, W, b, tab_dd, tab_plate, tab_magtype, tab_pos):
    x2d = x.reshape(_ROWS, _F)
    b2d = b.reshape(1, _D6)
    tcat = jnp.zeros((_D9, 3 * _D9), dtype=jnp.float32)
    tcat = tcat.at[0:20, 0:_D9].set(tab_dd)
    tcat = tcat.at[20:84, _D9:2 * _D9].set(tab_plate)
    tcat = tcat.at[84:104, 2 * _D9:3 * _D9].set(tab_magtype)

    grid = (_ROWS // _BLOCK_ROWS,)
    out = pl.pallas_call(
        _pe_kernel,
        grid=grid,
        in_specs=[
            pl.BlockSpec((_BLOCK_ROWS, _F), lambda i: (i, 0)),
            pl.BlockSpec((_F - 3, _D6), lambda i: (0, 0)),
            pl.BlockSpec((1, _D6), lambda i: (0, 0)),
            pl.BlockSpec((_D9, 3 * _D9), lambda i: (0, 0)),
            pl.BlockSpec((_S, _DM), lambda i: (0, 0)),
        ],
        out_specs=pl.BlockSpec((_BLOCK_ROWS, _DM), lambda i: (i, 0)),
        out_shape=jax.ShapeDtypeStruct((_ROWS, _DM), jnp.float32),
        compiler_params=pltpu.CompilerParams(
            dimension_semantics=("arbitrary",)),
    )(x2d, W, b2d, tcat, tab_pos)
    return out.reshape(_B, _S, _DM)
